# baseline (speedup 1.0000x reference)
import functools

import jax
import jax.numpy as jnp
from jax import lax
from jax.experimental import pallas as pl
from jax.experimental.pallas import tpu as pltpu
from jax.experimental.pallas import tpu_sc as plsc

PAIR = 1
GP = 8


def _make_gather(nw, nc, b, f, d):
    bpw = b // nw
    lp = PAIR * f
    pairs = bpw // PAIR
    groups = pairs // GP
    mesh = plsc.VectorSubcoreMesh(core_axis_name="c", subcore_axis_name="s")

    @functools.partial(
        pl.kernel,
        out_type=jax.ShapeDtypeStruct((b, f, d), jnp.float32),
        mesh=mesh,
        scratch_types=[
            pltpu.VMEM((bpw, f), jnp.int32),
            pltpu.VMEM((2, GP, lp, d), jnp.float32),
            pltpu.SemaphoreType.DMA,
        ],
        compiler_params=pltpu.CompilerParams(use_tc_tiling_on_sc=False),
    )
    def gather_kernel(idx_hbm, table_hbm, out_hbm, idx_v, rows_v, sem):
        wid = lax.axis_index("s") * nc + lax.axis_index("c")
        b0 = wid * bpw
        pltpu.sync_copy(idx_hbm.at[pl.ds(b0, bpw)], idx_v)

        def descs(g, slot):
            return [
                pltpu.make_async_copy(
                    table_hbm.at[idx_v.at[g * GP + j]],
                    rows_v.at[slot, j],
                    sem,
                )
                for j in range(GP)
            ]

        def fire(g, slot):
            for c in descs(g, slot):
                c.start()

        fire(0, 0)

        def body(g, carry):
            slot = lax.rem(g, 2)

            @pl.when(g + 1 < groups)
            def _():
                fire(g + 1, 1 - slot)

            for c in descs(g, slot):
                c.wait()
            for j in range(GP):
                for p in range(PAIR):
                    pltpu.sync_copy(
                        rows_v.at[slot, j, pl.ds(p * f, f)],
                        out_hbm.at[b0 + (g * GP + j) * PAIR + p],
                    )
            return carry

        lax.fori_loop(0, groups, body, 0)

    return gather_kernel


def kernel(indices, weight):
    b, f = indices.shape
    v, d = weight.shape
    info = plsc.get_sparse_core_info()
    nc, ns = info.num_cores, info.num_subcores
    nw = nc * ns
    assert b % (nw * PAIR * GP) == 0
    out = _make_gather(nw, nc, b, f, d)(indices, weight)
    return out


# speedup vs baseline: 1.1164x; 1.1164x over previous
import functools, jax, jax.numpy as jnp
from jax import lax
from jax.experimental import pallas as pl
from jax.experimental.pallas import tpu as pltpu, tpu_sc as plsc


def _mk():
    mesh = plsc.VectorSubcoreMesh(core_axis_name="c", subcore_axis_name="s")

    @functools.partial(
        pl.kernel,
        out_type=jax.ShapeDtypeStruct((4096, 100, 32), jnp.float32),
        mesh=mesh,
        scratch_types=[
            pltpu.VMEM((128, 112), jnp.int32),
            pltpu.VMEM((2, 100, 32), jnp.float32),
            pltpu.SemaphoreType.DMA,
            pltpu.SemaphoreType.DMA,
        ],
    )
    def k(idx_hbm, table_hbm, out_hbm, idx_v, rows_v, s0, s1):
        wid = lax.axis_index("s") * 2 + lax.axis_index("c")
        b0 = wid * 128

        def stage(bl, c):
            pltpu.make_async_copy(idx_hbm.at[b0 + bl], idx_v.at[bl, pl.ds(0, 100)], s0).start()
            return c
        lax.fori_loop(0, 128, stage, 0)

        def stwait(bl, c):
            pltpu.make_async_copy(idx_hbm.at[b0 + bl], idx_v.at[bl, pl.ds(0, 100)], s0).wait()
            return c
        lax.fori_loop(0, 128, stwait, 0)

        def body(bl, c):
            slot = lax.rem(bl, 2)
            for r0 in range(0, 112, 16):
                vec = idx_v[bl, pl.ds(r0, 16)]
                for l in range(16):
                    if r0 + l < 100:
                        pltpu.make_async_copy(table_hbm.at[vec[l]], rows_v.at[slot, r0 + l], s1).start()
            for r in range(100):
                pltpu.make_async_copy(table_hbm.at[0], rows_v.at[slot, r], s1).wait()
            pltpu.sync_copy(rows_v.at[slot], out_hbm.at[b0 + bl])
            return c
        lax.fori_loop(0, 128, body, 0)

    return k


def kernel(indices, weight):
    return _mk()(indices, weight)


# speedup vs baseline: 1.2841x; 1.1502x over previous
import functools

import jax
import jax.numpy as jnp
from jax import lax
from jax.experimental import pallas as pl
from jax.experimental.pallas import tpu as pltpu
from jax.experimental.pallas import tpu_sc as plsc

L = 16


def _make_lookup(nw, nc, b, f, d):
    bpw = b // nw
    fpad = ((f + L - 1) // L) * L
    mesh = plsc.VectorSubcoreMesh(core_axis_name="c", subcore_axis_name="s")

    @functools.partial(
        pl.kernel,
        out_type=jax.ShapeDtypeStruct((b, f, d), jnp.float32),
        mesh=mesh,
        scratch_types=[
            pltpu.VMEM((bpw, fpad), jnp.int32),
            pltpu.VMEM((2, f, d), jnp.float32),
            pltpu.SemaphoreType.DMA,
            pltpu.SemaphoreType.DMA,
            pltpu.SemaphoreType.DMA,
        ],
    )
    def lookup_kernel(idx_hbm, table_hbm, out_hbm, idx_v, rows_v, si, s0, s1):
        wid = lax.axis_index("s") * nc + lax.axis_index("c")
        b0 = wid * bpw

        def stage(bl, c):
            pltpu.make_async_copy(
                idx_hbm.at[b0 + bl], idx_v.at[bl, pl.ds(0, f)], si
            ).start()
            return c

        lax.fori_loop(0, bpw, stage, 0)

        def stage_wait(bl, c):
            pltpu.make_async_copy(
                idx_hbm.at[b0 + bl], idx_v.at[bl, pl.ds(0, f)], si
            ).wait()
            return c

        lax.fori_loop(0, bpw, stage_wait, 0)

        def fire(bl, slot, sem):
            for r0 in range(0, fpad, L):
                vec = idx_v[bl, pl.ds(r0, L)]
                for l in range(L):
                    if r0 + l < f:
                        pltpu.make_async_copy(
                            table_hbm.at[vec[l]], rows_v.at[slot, r0 + l], sem
                        ).start()

        def drain_write(bl, slot, sem):
            for r in range(f):
                pltpu.make_async_copy(
                    table_hbm.at[0], rows_v.at[slot, r], sem
                ).wait()
            pltpu.sync_copy(rows_v.at[slot], out_hbm.at[b0 + bl])

        fire(0, 0, s0)

        def body(blp, c):
            bl0 = 2 * blp
            fire(bl0 + 1, 1, s1)
            drain_write(bl0, 0, s0)

            @pl.when(bl0 + 2 < bpw)
            def _():
                fire(bl0 + 2, 0, s0)

            drain_write(bl0 + 1, 1, s1)
            return c

        lax.fori_loop(0, bpw // 2, body, 0)

    return lookup_kernel


def kernel(indices, weight):
    b, f = indices.shape
    v, d = weight.shape
    info = plsc.get_sparse_core_info()
    nc, ns = info.num_cores, info.num_subcores
    nw = nc * ns
    assert b % (2 * nw) == 0
    return _make_lookup(nw, nc, b, f, d)(indices, weight)
